# baseline (speedup 1.0000x reference)
import functools

import jax
import jax.numpy as jnp
from jax import lax
from jax.experimental import pallas as pl
from jax.experimental.pallas import tpu as pltpu
from jax.experimental.pallas import tpu_sc as plsc

N = 10000
E = 320000
D = 128
CLASSES = 40

N_PAD = 10112
N_SC = 2
N_TILES = 16
NW = N_SC * N_TILES
CHUNK = 128
EPW = E // NW
CHUNKS_PER_W = 78
TAIL = EPW - CHUNKS_PER_W * CHUNK
RPT = N_PAD // N_TILES

W2 = 48
WD = 16

_MESH = plsc.VectorSubcoreMesh(core_axis_name="c", subcore_axis_name="s")


def _make_agg(width, tc_tiling):
  @functools.partial(
      pl.kernel,
      out_type=jax.ShapeDtypeStruct((N_SC * N_PAD, width), jnp.float32),
      mesh=_MESH,
      compiler_params=pltpu.CompilerParams(use_tc_tiling_on_sc=tc_tiling),
      scratch_types=[
          [pltpu.VMEM((CHUNK,), jnp.int32)] * 3,
          [pltpu.VMEM((CHUNK,), jnp.int32)] * 3,
          [pltpu.VMEM((CHUNK, width), jnp.float32)] * 3,
          pltpu.VMEM((TAIL,), jnp.int32),
          pltpu.VMEM_SHARED((N_PAD, width), jnp.float32),
          pltpu.SemaphoreType.DMA,
          pltpu.SemaphoreType.DMA,
          pltpu.SemaphoreType.DMA,
          pltpu.SemaphoreType.DMA,
      ],
  )
  def agg(t_hbm, ei_hbm, zeros_hbm, out_hbm,
          srcb, dstb, rowsb, dtail, acc_sh, gsem, ssem, dsem, ksem):
    c = lax.axis_index("c")
    s = lax.axis_index("s")
    wid = c * N_TILES + s
    r0 = s * RPT
    e0 = wid * EPW
    pltpu.sync_copy(zeros_hbm.at[pl.ds(r0, RPT)], acc_sh.at[pl.ds(r0, RPT)])
    plsc.subcore_barrier()

    def idxload(i, b):
      pltpu.async_copy(ei_hbm.at[pl.ds(e0 + i * CHUNK, CHUNK)], srcb[b], ksem)
      pltpu.async_copy(ei_hbm.at[pl.ds(E + e0 + i * CHUNK, CHUNK)], dstb[b], dsem)

    def wait_idx(b):
      pltpu.make_async_copy(ei_hbm.at[pl.ds(e0, CHUNK)], srcb[b], ksem).wait()
      pltpu.make_async_copy(ei_hbm.at[pl.ds(e0, CHUNK)], dstb[b], dsem).wait()

    def gather(b):
      pltpu.async_copy(t_hbm.at[srcb[b]], rowsb[b], gsem)

    def wait_gather(b):
      pltpu.make_async_copy(t_hbm.at[srcb[0]], rowsb[b], gsem).wait()

    def wait_scatter():
      pltpu.make_async_copy(rowsb[0], acc_sh.at[dstb[0]], ssem).wait()

    idxload(0, 0)
    idxload(1, 1)
    wait_idx(0)
    gather(0)

    def body(g, carry):
      for b in range(3):
        i = 3 * g + b

        @pl.when(i + 1 < CHUNKS_PER_W)
        def _():
          wait_idx((b + 1) % 3)
          gather((b + 1) % 3)

        wait_gather(b)
        pltpu.async_copy(rowsb[b], acc_sh.at[dstb[b]], ssem, add=True)

        @pl.when(i >= 1)
        def _():
          wait_scatter()

        @pl.when(i + 2 < CHUNKS_PER_W)
        def _():
          idxload(i + 2, (b + 2) % 3)
      return carry

    lax.fori_loop(0, CHUNKS_PER_W // 3, body, 0)
    wait_scatter()
    et = e0 + CHUNKS_PER_W * CHUNK
    pltpu.async_copy(ei_hbm.at[pl.ds(et, TAIL)],
                     srcb[0].at[pl.ds(0, TAIL)], ksem)
    pltpu.async_copy(ei_hbm.at[pl.ds(E + et, TAIL)], dtail, dsem)
    pltpu.make_async_copy(ei_hbm.at[pl.ds(e0, TAIL)],
                          srcb[0].at[pl.ds(0, TAIL)], ksem).wait()
    pltpu.make_async_copy(ei_hbm.at[pl.ds(e0, TAIL)], dtail, dsem).wait()
    pltpu.async_copy(t_hbm.at[srcb[0].at[pl.ds(0, TAIL)]],
                     rowsb[0].at[pl.ds(0, TAIL)], gsem)
    pltpu.make_async_copy(t_hbm.at[srcb[0].at[pl.ds(0, TAIL)]],
                          rowsb[0].at[pl.ds(0, TAIL)], gsem).wait()
    pltpu.sync_copy(rowsb[0].at[pl.ds(0, TAIL)], acc_sh.at[dtail], add=True)
    plsc.subcore_barrier()
    pltpu.sync_copy(acc_sh.at[pl.ds(r0, RPT)],
                    out_hbm.at[pl.ds(c * N_PAD + r0, RPT)])

  return agg


@functools.partial(
    pl.kernel,
    out_type=jax.ShapeDtypeStruct((N_SC * N_PAD, WD), jnp.float32),
    mesh=_MESH,
    compiler_params=pltpu.CompilerParams(use_tc_tiling_on_sc=False),
    scratch_types=[
        pltpu.VMEM((CHUNK, WD), jnp.float32),
        [pltpu.VMEM((CHUNK,), jnp.int32)] * 3,
        pltpu.VMEM((TAIL,), jnp.int32),
        pltpu.VMEM_SHARED((N_PAD, WD), jnp.float32),
        pltpu.SemaphoreType.DMA,
        pltpu.SemaphoreType.DMA,
    ],
)
def _deg(ones_hbm, ei_hbm, zeros_hbm, out_hbm,
         ones_v, dstb, dtail, acc_sh, dsem, ssem):
  c = lax.axis_index("c")
  s = lax.axis_index("s")
  wid = c * N_TILES + s
  r0 = s * RPT
  e0 = wid * EPW
  pltpu.sync_copy(zeros_hbm.at[pl.ds(r0, RPT)], acc_sh.at[pl.ds(r0, RPT)])
  pltpu.sync_copy(ones_hbm, ones_v)
  plsc.subcore_barrier()

  def dstload(i, b):
    pltpu.async_copy(ei_hbm.at[pl.ds(E + e0 + i * CHUNK, CHUNK)], dstb[b], dsem)

  def wait_dstload(b):
    pltpu.make_async_copy(ei_hbm.at[pl.ds(e0, CHUNK)], dstb[b], dsem).wait()

  def wait_scatter():
    pltpu.make_async_copy(ones_v, acc_sh.at[dstb[0]], ssem).wait()

  dstload(0, 0)
  dstload(1, 1)

  def body(g, carry):
    for b in range(3):
      i = 3 * g + b
      wait_dstload(b)
      pltpu.async_copy(ones_v, acc_sh.at[dstb[b]], ssem, add=True)

      @pl.when(i >= 1)
      def _():
        wait_scatter()

      @pl.when(i + 2 < CHUNKS_PER_W)
      def _():
        dstload(i + 2, (b + 2) % 3)
    return carry

  lax.fori_loop(0, CHUNKS_PER_W // 3, body, 0)
  wait_scatter()
  et = e0 + CHUNKS_PER_W * CHUNK
  pltpu.async_copy(ei_hbm.at[pl.ds(E + et, TAIL)], dtail, dsem)
  pltpu.make_async_copy(ei_hbm.at[pl.ds(e0, TAIL)], dtail, dsem).wait()
  pltpu.sync_copy(ones_v.at[pl.ds(0, TAIL)], acc_sh.at[dtail], add=True)
  plsc.subcore_barrier()
  pltpu.sync_copy(acc_sh.at[pl.ds(r0, RPT)],
                  out_hbm.at[pl.ds(c * N_PAD + r0, RPT)])


_agg0 = _make_agg(D, True)
_agg2 = _make_agg(W2, False)


_R = 10112
_G = N_PAD // _R


def _mm0(x_pad, wn0):
  def body(x_ref, w_ref, o_ref):
    o_ref[...] = jnp.dot(x_ref[...], w_ref[...],
                         preferred_element_type=jnp.float32)

  return pl.pallas_call(
      body,
      grid=(_G,),
      in_specs=[pl.BlockSpec((_R, D), lambda i: (i, 0)),
                pl.BlockSpec((D, D), lambda i: (0, 0))],
      out_specs=pl.BlockSpec((_R, D), lambda i: (i, 0)),
      out_shape=jax.ShapeDtypeStruct((N_PAD, D), jnp.float32),
  )(x_pad, wn0)


def _combine0(x_pad, s0, sdeg, wself0, b0, wneigh1):
  def body(x_ref, sa_ref, sb_ref, da_ref, db_ref, ws_ref, b_ref, wn_ref,
           h1_ref, t1_ref, rdeg_ref):
    deg = da_ref[...] + db_ref[...]
    rdeg = 1.0 / jnp.maximum(deg, 1.0)
    neigh = (sa_ref[...] + sb_ref[...]) * rdeg
    h1 = jnp.maximum(
        jnp.dot(x_ref[...], ws_ref[...], preferred_element_type=jnp.float32)
        + neigh + b_ref[...], 0.0)
    h1_ref[...] = h1
    t1_ref[...] = jnp.dot(h1, wn_ref[...], preferred_element_type=jnp.float32)
    rdeg_ref[...] = rdeg

  return pl.pallas_call(
      body,
      grid=(_G,),
      in_specs=[
          pl.BlockSpec((_R, D), lambda i: (i, 0)),
          pl.BlockSpec((_R, D), lambda i: (i, 0)),
          pl.BlockSpec((_R, D), lambda i: (i + _G, 0)),
          pl.BlockSpec((_R, 1), lambda i: (i, 0)),
          pl.BlockSpec((_R, 1), lambda i: (i + _G, 0)),
          pl.BlockSpec((D, D), lambda i: (0, 0)),
          pl.BlockSpec((1, D), lambda i: (0, 0)),
          pl.BlockSpec((D, D), lambda i: (0, 0)),
      ],
      out_specs=[
          pl.BlockSpec((_R, D), lambda i: (i, 0)),
          pl.BlockSpec((_R, D), lambda i: (i, 0)),
          pl.BlockSpec((_R, 1), lambda i: (i, 0)),
      ],
      out_shape=[
          jax.ShapeDtypeStruct((N_PAD, D), jnp.float32),
          jax.ShapeDtypeStruct((N_PAD, D), jnp.float32),
          jax.ShapeDtypeStruct((N_PAD, 1), jnp.float32),
      ],
  )(x_pad, s0, s0, sdeg, sdeg, wself0, b0, wneigh1)


def _combine1(h1, s1, rdeg, wself1, b1, wneigh2_pad, wself2):
  def body(h_ref, sa_ref, sb_ref, rd_ref, ws_ref, b_ref, wn_ref, w2_ref,
           t2_ref, u2_ref):
    neigh = (sa_ref[...] + sb_ref[...]) * rd_ref[...]
    h2 = jnp.maximum(
        jnp.dot(h_ref[...], ws_ref[...], preferred_element_type=jnp.float32)
        + neigh + b_ref[...], 0.0)
    t2_ref[...] = jnp.dot(h2, wn_ref[...], preferred_element_type=jnp.float32)
    u2_ref[...] = jnp.dot(h2, w2_ref[...], preferred_element_type=jnp.float32)

  return pl.pallas_call(
      body,
      grid=(_G,),
      in_specs=[
          pl.BlockSpec((_R, D), lambda i: (i, 0)),
          pl.BlockSpec((_R, D), lambda i: (i, 0)),
          pl.BlockSpec((_R, D), lambda i: (i + _G, 0)),
          pl.BlockSpec((_R, 1), lambda i: (i, 0)),
          pl.BlockSpec((D, D), lambda i: (0, 0)),
          pl.BlockSpec((1, D), lambda i: (0, 0)),
          pl.BlockSpec((D, W2), lambda i: (0, 0)),
          pl.BlockSpec((D, CLASSES), lambda i: (0, 0)),
      ],
      out_specs=[
          pl.BlockSpec((_R, W2), lambda i: (i, 0)),
          pl.BlockSpec((_R, CLASSES), lambda i: (i, 0)),
      ],
      out_shape=[
          jax.ShapeDtypeStruct((N_PAD, W2), jnp.float32),
          jax.ShapeDtypeStruct((N_PAD, CLASSES), jnp.float32),
      ],
  )(h1, s1, s1, rdeg, wself1, b1, wneigh2_pad, wself2)


def _combine2(u2, s2, rdeg, b2):
  def body(u_ref, sa_ref, sb_ref, rd_ref, b_ref, o_ref):
    sm = (sa_ref[...] + sb_ref[...])[:, :CLASSES]
    o_ref[...] = u_ref[...] + sm * rd_ref[...] + b_ref[...]

  return pl.pallas_call(
      body,
      grid=(_G,),
      in_specs=[
          pl.BlockSpec((_R, CLASSES), lambda i: (i, 0)),
          pl.BlockSpec((_R, W2), lambda i: (i, 0)),
          pl.BlockSpec((_R, W2), lambda i: (i + _G, 0)),
          pl.BlockSpec((_R, 1), lambda i: (i, 0)),
          pl.BlockSpec((1, CLASSES), lambda i: (0, 0)),
      ],
      out_specs=pl.BlockSpec((_R, CLASSES), lambda i: (i, 0)),
      out_shape=jax.ShapeDtypeStruct((N, CLASSES), jnp.float32),
  )(u2, s2, s2, rdeg, b2)


def kernel(x, edge_index, Wself0, Wneigh0, b0, Wself1, Wneigh1, b1,
           Wself2, Wneigh2, b2):
  x_pad = jnp.pad(x, ((0, N_PAD - N), (0, 0)))
  wn2_pad = jnp.pad(Wneigh2, ((0, 0), (0, W2 - CLASSES)))
  z128 = jnp.zeros((N_PAD, D), jnp.float32)
  z48 = jnp.zeros((N_PAD, W2), jnp.float32)
  z16 = jnp.zeros((N_PAD, WD), jnp.float32)
  ones16 = jnp.ones((CHUNK, WD), jnp.float32)

  ei_flat = edge_index.reshape(2 * E)
  sdeg = _deg(ones16, ei_flat, z16)
  deg2 = sdeg[:, 0:1]
  t0 = _mm0(x_pad, Wneigh0)
  s0 = _agg0(t0, ei_flat, z128)
  h1, t1, rdeg = _combine0(x_pad, s0, deg2, Wself0, b0.reshape(1, D), Wneigh1)
  s1 = _agg0(t1, ei_flat, z128)
  t2, u2 = _combine1(h1, s1, rdeg, Wself1, b1.reshape(1, D), wn2_pad, Wself2)
  s2 = _agg2(t2, ei_flat, z48)
  return _combine2(u2, s2, rdeg, b2.reshape(1, CLASSES))


# speedup vs baseline: 1.0165x; 1.0165x over previous
import functools

import jax
import jax.numpy as jnp
from jax import lax
from jax.experimental import pallas as pl
from jax.experimental.pallas import tpu as pltpu
from jax.experimental.pallas import tpu_sc as plsc

N = 10000
E = 320000
D = 128
CLASSES = 40

N_PAD = 10112
N_SC = 2
N_TILES = 16
NW = N_SC * N_TILES
CHUNK = 128
EPW = E // NW
CHUNKS_PER_W = 78
TAIL = EPW - CHUNKS_PER_W * CHUNK
RPT = N_PAD // N_TILES

W2 = 48
WD = 16

_MESH = plsc.VectorSubcoreMesh(core_axis_name="c", subcore_axis_name="s")


def _make_agg(width, tc_tiling):
  @functools.partial(
      pl.kernel,
      out_type=jax.ShapeDtypeStruct((N_SC * N_PAD, width), jnp.float32),
      mesh=_MESH,
      compiler_params=pltpu.CompilerParams(use_tc_tiling_on_sc=tc_tiling),
      scratch_types=[
          [pltpu.VMEM((CHUNK,), jnp.int32)] * 3,
          [pltpu.VMEM((CHUNK,), jnp.int32)] * 3,
          [pltpu.VMEM((CHUNK, width), jnp.float32)] * 3,
          pltpu.VMEM((TAIL,), jnp.int32),
          pltpu.VMEM_SHARED((N_PAD, width), jnp.float32),
          pltpu.SemaphoreType.DMA,
          pltpu.SemaphoreType.DMA,
          pltpu.SemaphoreType.DMA,
          pltpu.SemaphoreType.DMA,
      ],
  )
  def agg(t_hbm, ei_hbm, zeros_hbm, out_hbm,
          srcb, dstb, rowsb, dtail, acc_sh, gsem, ssem, dsem, ksem):
    c = lax.axis_index("c")
    s = lax.axis_index("s")
    wid = c * N_TILES + s
    r0 = s * RPT
    e0 = wid * EPW
    pltpu.sync_copy(zeros_hbm.at[pl.ds(r0, RPT)], acc_sh.at[pl.ds(r0, RPT)])
    plsc.subcore_barrier()

    def idxload(i, b):
      pltpu.async_copy(ei_hbm.at[pl.ds(e0 + i * CHUNK, CHUNK)], srcb[b], ksem)
      pltpu.async_copy(ei_hbm.at[pl.ds(E + e0 + i * CHUNK, CHUNK)], dstb[b], dsem)

    def wait_idx(b):
      pltpu.make_async_copy(ei_hbm.at[pl.ds(e0, CHUNK)], srcb[b], ksem).wait()
      pltpu.make_async_copy(ei_hbm.at[pl.ds(e0, CHUNK)], dstb[b], dsem).wait()

    def gather(b):
      pltpu.async_copy(t_hbm.at[srcb[b]], rowsb[b], gsem)

    def wait_gather(b):
      pltpu.make_async_copy(t_hbm.at[srcb[0]], rowsb[b], gsem).wait()

    def wait_scatter():
      pltpu.make_async_copy(rowsb[0], acc_sh.at[dstb[0]], ssem).wait()

    idxload(0, 0)
    idxload(1, 1)
    wait_idx(0)
    gather(0)

    def body(g, carry):
      for b in range(3):
        i = 3 * g + b

        @pl.when(i + 1 < CHUNKS_PER_W)
        def _():
          wait_idx((b + 1) % 3)
          gather((b + 1) % 3)

        wait_gather(b)
        pltpu.async_copy(rowsb[b], acc_sh.at[dstb[b]], ssem, add=True)

        @pl.when(i >= 1)
        def _():
          wait_scatter()

        @pl.when(i + 2 < CHUNKS_PER_W)
        def _():
          idxload(i + 2, (b + 2) % 3)
      return carry

    lax.fori_loop(0, CHUNKS_PER_W // 3, body, 0)
    wait_scatter()
    et = e0 + CHUNKS_PER_W * CHUNK
    pltpu.async_copy(ei_hbm.at[pl.ds(et, TAIL)],
                     srcb[0].at[pl.ds(0, TAIL)], ksem)
    pltpu.async_copy(ei_hbm.at[pl.ds(E + et, TAIL)], dtail, dsem)
    pltpu.make_async_copy(ei_hbm.at[pl.ds(e0, TAIL)],
                          srcb[0].at[pl.ds(0, TAIL)], ksem).wait()
    pltpu.make_async_copy(ei_hbm.at[pl.ds(e0, TAIL)], dtail, dsem).wait()
    pltpu.async_copy(t_hbm.at[srcb[0].at[pl.ds(0, TAIL)]],
                     rowsb[0].at[pl.ds(0, TAIL)], gsem)
    pltpu.make_async_copy(t_hbm.at[srcb[0].at[pl.ds(0, TAIL)]],
                          rowsb[0].at[pl.ds(0, TAIL)], gsem).wait()
    pltpu.sync_copy(rowsb[0].at[pl.ds(0, TAIL)], acc_sh.at[dtail], add=True)
    plsc.subcore_barrier()
    pltpu.sync_copy(acc_sh.at[pl.ds(r0, RPT)],
                    out_hbm.at[pl.ds(c * N_PAD + r0, RPT)])

  return agg


@functools.partial(
    pl.kernel,
    out_type=jax.ShapeDtypeStruct((N_SC * N_PAD, WD), jnp.float32),
    mesh=_MESH,
    compiler_params=pltpu.CompilerParams(use_tc_tiling_on_sc=False),
    scratch_types=[
        pltpu.VMEM((CHUNK, WD), jnp.float32),
        [pltpu.VMEM((CHUNK,), jnp.int32)] * 3,
        pltpu.VMEM((TAIL,), jnp.int32),
        pltpu.VMEM_SHARED((N_PAD, WD), jnp.float32),
        pltpu.SemaphoreType.DMA,
        pltpu.SemaphoreType.DMA,
    ],
)
def _deg(ones_hbm, ei_hbm, zeros_hbm, out_hbm,
         ones_v, dstb, dtail, acc_sh, dsem, ssem):
  c = lax.axis_index("c")
  s = lax.axis_index("s")
  wid = c * N_TILES + s
  r0 = s * RPT
  e0 = wid * EPW
  pltpu.sync_copy(zeros_hbm.at[pl.ds(r0, RPT)], acc_sh.at[pl.ds(r0, RPT)])
  pltpu.sync_copy(ones_hbm, ones_v)
  plsc.subcore_barrier()

  def dstload(i, b):
    pltpu.async_copy(ei_hbm.at[pl.ds(E + e0 + i * CHUNK, CHUNK)], dstb[b], dsem)

  def wait_dstload(b):
    pltpu.make_async_copy(ei_hbm.at[pl.ds(e0, CHUNK)], dstb[b], dsem).wait()

  def wait_scatter():
    pltpu.make_async_copy(ones_v, acc_sh.at[dstb[0]], ssem).wait()

  dstload(0, 0)
  dstload(1, 1)

  def body(g, carry):
    for b in range(3):
      i = 3 * g + b
      wait_dstload(b)
      pltpu.async_copy(ones_v, acc_sh.at[dstb[b]], ssem, add=True)

      @pl.when(i >= 1)
      def _():
        wait_scatter()

      @pl.when(i + 2 < CHUNKS_PER_W)
      def _():
        dstload(i + 2, (b + 2) % 3)
    return carry

  lax.fori_loop(0, CHUNKS_PER_W // 3, body, 0)
  wait_scatter()
  et = e0 + CHUNKS_PER_W * CHUNK
  pltpu.async_copy(ei_hbm.at[pl.ds(E + et, TAIL)], dtail, dsem)
  pltpu.make_async_copy(ei_hbm.at[pl.ds(e0, TAIL)], dtail, dsem).wait()
  pltpu.sync_copy(ones_v.at[pl.ds(0, TAIL)], acc_sh.at[dtail], add=True)
  plsc.subcore_barrier()
  pltpu.sync_copy(acc_sh.at[pl.ds(r0, RPT)],
                  out_hbm.at[pl.ds(c * N_PAD + r0, RPT)])


_agg0 = _make_agg(D, True)
_agg2 = _make_agg(W2, False)


_R = 5056
_G = N_PAD // _R


def _mm0(x_pad, wn0):
  def body(x_ref, w_ref, o_ref):
    o_ref[...] = jnp.dot(x_ref[...], w_ref[...],
                         preferred_element_type=jnp.float32)

  return pl.pallas_call(
      body,
      grid=(_G,),
      in_specs=[pl.BlockSpec((_R, D), lambda i: (i, 0)),
                pl.BlockSpec((D, D), lambda i: (0, 0))],
      out_specs=pl.BlockSpec((_R, D), lambda i: (i, 0)),
      out_shape=jax.ShapeDtypeStruct((N_PAD, D), jnp.float32),
  )(x_pad, wn0)


def _combine0(x_pad, s0, sdeg, wself0, b0, wneigh1):
  def body(x_ref, sa_ref, sb_ref, da_ref, db_ref, ws_ref, b_ref, wn_ref,
           h1_ref, t1_ref, rdeg_ref):
    deg = da_ref[...] + db_ref[...]
    rdeg = 1.0 / jnp.maximum(deg, 1.0)
    neigh = (sa_ref[...] + sb_ref[...]) * rdeg
    h1 = jnp.maximum(
        jnp.dot(x_ref[...], ws_ref[...], preferred_element_type=jnp.float32)
        + neigh + b_ref[...], 0.0)
    h1_ref[...] = h1
    t1_ref[...] = jnp.dot(h1, wn_ref[...], preferred_element_type=jnp.float32)
    rdeg_ref[...] = rdeg

  return pl.pallas_call(
      body,
      grid=(_G,),
      in_specs=[
          pl.BlockSpec((_R, D), lambda i: (i, 0)),
          pl.BlockSpec((_R, D), lambda i: (i, 0)),
          pl.BlockSpec((_R, D), lambda i: (i + _G, 0)),
          pl.BlockSpec((_R, 1), lambda i: (i, 0)),
          pl.BlockSpec((_R, 1), lambda i: (i + _G, 0)),
          pl.BlockSpec((D, D), lambda i: (0, 0)),
          pl.BlockSpec((1, D), lambda i: (0, 0)),
          pl.BlockSpec((D, D), lambda i: (0, 0)),
      ],
      out_specs=[
          pl.BlockSpec((_R, D), lambda i: (i, 0)),
          pl.BlockSpec((_R, D), lambda i: (i, 0)),
          pl.BlockSpec((_R, 1), lambda i: (i, 0)),
      ],
      out_shape=[
          jax.ShapeDtypeStruct((N_PAD, D), jnp.float32),
          jax.ShapeDtypeStruct((N_PAD, D), jnp.float32),
          jax.ShapeDtypeStruct((N_PAD, 1), jnp.float32),
      ],
  )(x_pad, s0, s0, sdeg, sdeg, wself0, b0, wneigh1)


def _combine1(h1, s1, rdeg, wself1, b1, wneigh2_pad, wself2):
  def body(h_ref, sa_ref, sb_ref, rd_ref, ws_ref, b_ref, wn_ref, w2_ref,
           t2_ref, u2_ref):
    neigh = (sa_ref[...] + sb_ref[...]) * rd_ref[...]
    h2 = jnp.maximum(
        jnp.dot(h_ref[...], ws_ref[...], preferred_element_type=jnp.float32)
        + neigh + b_ref[...], 0.0)
    t2_ref[...] = jnp.dot(h2, wn_ref[...], preferred_element_type=jnp.float32)
    u2_ref[...] = jnp.dot(h2, w2_ref[...], preferred_element_type=jnp.float32)

  return pl.pallas_call(
      body,
      grid=(_G,),
      in_specs=[
          pl.BlockSpec((_R, D), lambda i: (i, 0)),
          pl.BlockSpec((_R, D), lambda i: (i, 0)),
          pl.BlockSpec((_R, D), lambda i: (i + _G, 0)),
          pl.BlockSpec((_R, 1), lambda i: (i, 0)),
          pl.BlockSpec((D, D), lambda i: (0, 0)),
          pl.BlockSpec((1, D), lambda i: (0, 0)),
          pl.BlockSpec((D, W2), lambda i: (0, 0)),
          pl.BlockSpec((D, CLASSES), lambda i: (0, 0)),
      ],
      out_specs=[
          pl.BlockSpec((_R, W2), lambda i: (i, 0)),
          pl.BlockSpec((_R, CLASSES), lambda i: (i, 0)),
      ],
      out_shape=[
          jax.ShapeDtypeStruct((N_PAD, W2), jnp.float32),
          jax.ShapeDtypeStruct((N_PAD, CLASSES), jnp.float32),
      ],
  )(h1, s1, s1, rdeg, wself1, b1, wneigh2_pad, wself2)


def _combine2(u2, s2, rdeg, b2):
  def body(u_ref, sa_ref, sb_ref, rd_ref, b_ref, o_ref):
    sm = (sa_ref[...] + sb_ref[...])[:, :CLASSES]
    o_ref[...] = u_ref[...] + sm * rd_ref[...] + b_ref[...]

  return pl.pallas_call(
      body,
      grid=(_G,),
      in_specs=[
          pl.BlockSpec((_R, CLASSES), lambda i: (i, 0)),
          pl.BlockSpec((_R, W2), lambda i: (i, 0)),
          pl.BlockSpec((_R, W2), lambda i: (i + _G, 0)),
          pl.BlockSpec((_R, 1), lambda i: (i, 0)),
          pl.BlockSpec((1, CLASSES), lambda i: (0, 0)),
      ],
      out_specs=pl.BlockSpec((_R, CLASSES), lambda i: (i, 0)),
      out_shape=jax.ShapeDtypeStruct((N, CLASSES), jnp.float32),
  )(u2, s2, s2, rdeg, b2)


def kernel(x, edge_index, Wself0, Wneigh0, b0, Wself1, Wneigh1, b1,
           Wself2, Wneigh2, b2):
  x_pad = jnp.pad(x, ((0, N_PAD - N), (0, 0)))
  wn2_pad = jnp.pad(Wneigh2, ((0, 0), (0, W2 - CLASSES)))
  z128 = jnp.zeros((N_PAD, D), jnp.float32)
  z48 = jnp.zeros((N_PAD, W2), jnp.float32)
  z16 = jnp.zeros((N_PAD, WD), jnp.float32)
  ones16 = jnp.ones((CHUNK, WD), jnp.float32)

  ei_flat = edge_index.reshape(2 * E)
  sdeg = _deg(ones16, ei_flat, z16)
  deg2 = sdeg[:, 0:1]
  t0 = _mm0(x_pad, Wneigh0)
  s0 = _agg0(t0, ei_flat, z128)
  h1, t1, rdeg = _combine0(x_pad, s0, deg2, Wself0, b0.reshape(1, D), Wneigh1)
  s1 = _agg0(t1, ei_flat, z128)
  t2, u2 = _combine1(h1, s1, rdeg, Wself1, b1.reshape(1, D), wn2_pad, Wself2)
  s2 = _agg2(t2, ei_flat, z48)
  return _combine2(u2, s2, rdeg, b2.reshape(1, CLASSES))
